# Initial kernel scaffold; baseline (speedup 1.0000x reference)
import jax
import jax.numpy as jnp
from jax.experimental import pallas as pl


def kernel(h, edge_index, rel_types, W1, W1_self, b1, W2, W2_self, b2, Wc, bc):
    raise NotImplementedError("write your pallas kernel here")


# speedup vs baseline: 6.7478x; 6.7478x over previous
import functools

import jax
import jax.numpy as jnp
from jax import lax
from jax.experimental import pallas as pl
from jax.experimental.pallas import tpu as pltpu
from jax.experimental.pallas import tpu_sc as plsc

_NC = 2
_NS = 16
_K = 128

_N = 10000
_NROWS = 10112
_ZR = _NROWS // _NS
_PAD_DST = _N
_OR_HI = 632
_OR_LO = _N - 15 * _OR_HI


@functools.lru_cache(maxsize=None)
def _make_sc_accum(n_chunks, table_rows):
    mesh = plsc.VectorSubcoreMesh(core_axis_name="c", subcore_axis_name="s")

    @functools.partial(
        pl.kernel,
        out_type=jax.ShapeDtypeStruct((_NC, _N, 128), jnp.float32),
        mesh=mesh,
        scratch_types=[
            pltpu.VMEM((_K,), jnp.int32),
            pltpu.VMEM((_K,), jnp.int32),
            pltpu.VMEM((_K, 128), jnp.float32),
            pltpu.VMEM_SHARED((_NROWS, 128), jnp.float32),
            pltpu.SemaphoreType.DMA,
        ],
    )
    def sc_accum(table, keys, dsts, zeros, out, keys_v, dst_v, rows_v, acc, sem):
        c = lax.axis_index("c")
        s = lax.axis_index("s")
        pltpu.sync_copy(zeros.at[pl.ds(s * _ZR, _ZR)], acc.at[pl.ds(s * _ZR, _ZR)])
        plsc.subcore_barrier()

        tile_base = (c * _NS + s) * n_chunks * _K

        @pl.loop(0, n_chunks)
        def _chunk(j):
            off = tile_base + j * _K
            pltpu.sync_copy(keys.at[pl.ds(off, _K)], keys_v)
            pltpu.sync_copy(dsts.at[pl.ds(off, _K)], dst_v)
            pltpu.async_copy(table.at[keys_v], rows_v, sem).wait()
            pltpu.sync_copy(rows_v, acc.at[dst_v], add=True)

        plsc.subcore_barrier()

        @pl.when(s < _NS - 1)
        def _copy_hi():
            pltpu.sync_copy(acc.at[pl.ds(s * _OR_HI, _OR_HI)],
                            out.at[c, pl.ds(s * _OR_HI, _OR_HI)])

        @pl.when(s == _NS - 1)
        def _copy_lo():
            pltpu.sync_copy(acc.at[pl.ds(15 * _OR_HI, _OR_LO)],
                            out.at[c, pl.ds(15 * _OR_HI, _OR_LO)])

    return sc_accum


def _mm_body(x_ref, w_ref, o_ref):
    o_ref[0] = jnp.dot(x_ref[...], w_ref[0], preferred_element_type=jnp.float32)


def _stacked_mm(x, w):
    g, din, dout = w.shape
    n = x.shape[0]
    bm = 1000
    return pl.pallas_call(
        _mm_body,
        grid=(g, n // bm),
        in_specs=[
            pl.BlockSpec((bm, din), lambda gi, m: (m, 0)),
            pl.BlockSpec((1, din, dout), lambda gi, m: (gi, 0, 0)),
        ],
        out_specs=pl.BlockSpec((1, bm, dout), lambda gi, m: (gi, m, 0)),
        out_shape=jax.ShapeDtypeStruct((g, n, dout), jnp.float32),
    )(x, w)


def _combine1_body(a_ref, b_ref, o_ref):
    bv = b_ref[...]
    o_ref[:, 0:128] = jnp.maximum(a_ref[0] + bv[:, 0:128], 0.0)
    o_ref[:, 128:256] = jnp.maximum(a_ref[1] + bv[:, 128:256], 0.0)


def _combine1(agg, b):
    bm = 1000
    return pl.pallas_call(
        _combine1_body,
        grid=(_N // bm,),
        in_specs=[
            pl.BlockSpec((2, bm, 128), lambda m: (0, m, 0)),
            pl.BlockSpec((1, 256), lambda m: (0, 0)),
        ],
        out_specs=pl.BlockSpec((bm, 256), lambda m: (m, 0)),
        out_shape=jax.ShapeDtypeStruct((_N, 256), jnp.float32),
    )(agg, b)


def _final_body(a_ref, b_ref, wc_ref, bc_ref, o_ref):
    h2 = jnp.maximum(a_ref[0] + a_ref[1] + b_ref[...], 0.0)
    m = jnp.sum(h2, axis=0, keepdims=True) * (1.0 / _N)
    logits = jnp.dot(m, wc_ref[...], preferred_element_type=jnp.float32) + bc_ref[...]
    z = logits - jnp.max(logits, axis=1, keepdims=True)
    e = jnp.exp(z)
    o_ref[...] = e / jnp.sum(e, axis=1, keepdims=True)


def _final(agg, b2, wc, bc):
    c = wc.shape[1]
    return pl.pallas_call(
        _final_body,
        out_shape=jax.ShapeDtypeStruct((1, c), jnp.float32),
    )(agg, b2, wc, bc)


def _pad_reshape(arr, per_core, n_chunks, pad_value):
    target = _NS * n_chunks * _K
    arr = jnp.pad(arr, ((0, 0), (0, target - per_core)), constant_values=pad_value)
    return arr.reshape(-1)


def kernel(h, edge_index, rel_types, W1, W1_self, b1, W2, W2_self, b2, Wc, bc):
    h = h.astype(jnp.float32)
    n = h.shape[0]
    nr = W1.shape[0]

    w1_all = jnp.concatenate([W1, W1_self[None]], axis=0)
    w2_all = jnp.concatenate([W2, W2_self[None]], axis=0)

    src = edge_index[0]
    dst = edge_index[1]
    nid = jnp.arange(n, dtype=jnp.int32)
    keys_all = jnp.concatenate([rel_types * n + src, nr * n + nid])
    dst_all = jnp.concatenate([dst, nid])
    e_tot = keys_all.shape[0]

    nch1 = -(-e_tot // (_NS * _K))
    k2 = keys_all * 2
    keys1 = _pad_reshape(jnp.stack([k2, k2 + 1]), e_tot, nch1, 0)
    dst1 = _pad_reshape(jnp.stack([dst_all, dst_all]), e_tot, nch1, _PAD_DST)

    eh = -(-e_tot // 2)
    nch2 = -(-eh // (_NS * _K))
    pad_tail = 2 * eh - e_tot
    keys_h = jnp.pad(keys_all, (0, pad_tail)).reshape(2, eh)
    dst_h = jnp.pad(dst_all, (0, pad_tail), constant_values=_PAD_DST).reshape(2, eh)
    keys2 = _pad_reshape(keys_h, eh, nch2, 0)
    dst2 = _pad_reshape(dst_h, eh, nch2, _PAD_DST)

    zeros = jnp.zeros((_NROWS, 128), jnp.float32)

    proj1 = _stacked_mm(h, w1_all)
    table1 = proj1.reshape((nr + 1) * n * 2, 128)
    agg1 = _make_sc_accum(nch1, table1.shape[0])(table1, keys1, dst1, zeros)
    h1 = _combine1(agg1, b1.reshape(1, 256))

    proj2 = _stacked_mm(h1, w2_all)
    table2 = proj2.reshape((nr + 1) * n, 128)
    agg2 = _make_sc_accum(nch2, table2.shape[0])(table2, keys2, dst2, zeros)

    return _final(agg2, b2.reshape(1, 128), Wc, bc.reshape(1, Wc.shape[1]))


# speedup vs baseline: 8.5652x; 1.2693x over previous
import functools

import jax
import jax.numpy as jnp
from jax import lax
from jax.experimental import pallas as pl
from jax.experimental.pallas import tpu as pltpu
from jax.experimental.pallas import tpu_sc as plsc

_NC = 2
_NS = 16
_K = 128

_N = 10000
_NROWS = 10112
_ZR = _NROWS // _NS
_PAD_DST = _N
_OR_HI = 632
_OR_LO = _N - 15 * _OR_HI


_NBUF = 3


@functools.lru_cache(maxsize=None)
def _make_sc_accum(n_chunks, table_rows):
    assert n_chunks % _NBUF == 0
    mesh = plsc.VectorSubcoreMesh(core_axis_name="c", subcore_axis_name="s")

    scratch = []
    for _ in range(_NBUF):
        scratch += [
            pltpu.VMEM((_K,), jnp.int32),
            pltpu.VMEM((_K,), jnp.int32),
            pltpu.VMEM((_K, 128), jnp.float32),
            pltpu.SemaphoreType.DMA,
        ]
    scratch.append(pltpu.VMEM_SHARED((_NROWS, 128), jnp.float32))

    @functools.partial(
        pl.kernel,
        out_type=jax.ShapeDtypeStruct((_NC, _N, 128), jnp.float32),
        mesh=mesh,
        scratch_types=scratch,
    )
    def sc_accum(table, keys, dsts, zeros, out, *refs):
        acc = refs[-1]
        kv = [refs[4 * b + 0] for b in range(_NBUF)]
        dv = [refs[4 * b + 1] for b in range(_NBUF)]
        rv = [refs[4 * b + 2] for b in range(_NBUF)]
        sem = [refs[4 * b + 3] for b in range(_NBUF)]
        c = lax.axis_index("c")
        s = lax.axis_index("s")
        pltpu.sync_copy(zeros.at[pl.ds(s * _ZR, _ZR)], acc.at[pl.ds(s * _ZR, _ZR)])
        plsc.subcore_barrier()

        tile_base = (c * _NS + s) * n_chunks * _K

        for b in range(_NBUF):
            off = tile_base + b * _K
            pltpu.sync_copy(keys.at[pl.ds(off, _K)], kv[b])
            pltpu.sync_copy(dsts.at[pl.ds(off, _K)], dv[b])
            pltpu.async_copy(table.at[kv[b]], rv[b], sem[b])

        @pl.loop(0, n_chunks, step=_NBUF)
        def _chunk(j):
            for b in range(_NBUF):
                pltpu.make_async_copy(table.at[kv[b]], rv[b], sem[b]).wait()
                pltpu.sync_copy(rv[b], acc.at[dv[b]], add=True)
                nxt = j + b + _NBUF

                @pl.when(nxt < n_chunks)
                def _prefetch():
                    off = tile_base + nxt * _K
                    pltpu.sync_copy(keys.at[pl.ds(off, _K)], kv[b])
                    pltpu.sync_copy(dsts.at[pl.ds(off, _K)], dv[b])
                    pltpu.async_copy(table.at[kv[b]], rv[b], sem[b])

        plsc.subcore_barrier()

        @pl.when(s < _NS - 1)
        def _copy_hi():
            pltpu.sync_copy(acc.at[pl.ds(s * _OR_HI, _OR_HI)],
                            out.at[c, pl.ds(s * _OR_HI, _OR_HI)])

        @pl.when(s == _NS - 1)
        def _copy_lo():
            pltpu.sync_copy(acc.at[pl.ds(15 * _OR_HI, _OR_LO)],
                            out.at[c, pl.ds(15 * _OR_HI, _OR_LO)])

    return sc_accum


def _mm_body(x_ref, w_ref, o_ref):
    o_ref[0] = jnp.dot(x_ref[...], w_ref[0], preferred_element_type=jnp.float32)


def _stacked_mm(x, w):
    g, din, dout = w.shape
    n = x.shape[0]
    bm = 1000
    return pl.pallas_call(
        _mm_body,
        grid=(g, n // bm),
        in_specs=[
            pl.BlockSpec((bm, din), lambda gi, m: (m, 0)),
            pl.BlockSpec((1, din, dout), lambda gi, m: (gi, 0, 0)),
        ],
        out_specs=pl.BlockSpec((1, bm, dout), lambda gi, m: (gi, m, 0)),
        out_shape=jax.ShapeDtypeStruct((g, n, dout), jnp.float32),
    )(x, w)


def _combine1_body(a_ref, b_ref, o_ref):
    bv = b_ref[...]
    o_ref[:, 0:128] = jnp.maximum(a_ref[0] + bv[:, 0:128], 0.0)
    o_ref[:, 128:256] = jnp.maximum(a_ref[1] + bv[:, 128:256], 0.0)


def _combine1(agg, b):
    bm = 1000
    return pl.pallas_call(
        _combine1_body,
        grid=(_N // bm,),
        in_specs=[
            pl.BlockSpec((2, bm, 128), lambda m: (0, m, 0)),
            pl.BlockSpec((1, 256), lambda m: (0, 0)),
        ],
        out_specs=pl.BlockSpec((bm, 256), lambda m: (m, 0)),
        out_shape=jax.ShapeDtypeStruct((_N, 256), jnp.float32),
    )(agg, b)


def _final_body(a_ref, b_ref, wc_ref, bc_ref, o_ref):
    h2 = jnp.maximum(a_ref[0] + a_ref[1] + b_ref[...], 0.0)
    m = jnp.sum(h2, axis=0, keepdims=True) * (1.0 / _N)
    logits = jnp.dot(m, wc_ref[...], preferred_element_type=jnp.float32) + bc_ref[...]
    z = logits - jnp.max(logits, axis=1, keepdims=True)
    e = jnp.exp(z)
    o_ref[...] = e / jnp.sum(e, axis=1, keepdims=True)


def _final(agg, b2, wc, bc):
    c = wc.shape[1]
    return pl.pallas_call(
        _final_body,
        out_shape=jax.ShapeDtypeStruct((1, c), jnp.float32),
    )(agg, b2, wc, bc)


def _round_up(x, m):
    return -(-x // m) * m


def _pad_reshape(arr, per_core, n_chunks, pad_value):
    target = _NS * n_chunks * _K
    arr = jnp.pad(arr, ((0, 0), (0, target - per_core)), constant_values=pad_value)
    return arr.reshape(-1)


def kernel(h, edge_index, rel_types, W1, W1_self, b1, W2, W2_self, b2, Wc, bc):
    h = h.astype(jnp.float32)
    n = h.shape[0]
    nr = W1.shape[0]

    w1_all = jnp.concatenate([W1, W1_self[None]], axis=0)
    w2_all = jnp.concatenate([W2, W2_self[None]], axis=0)

    src = edge_index[0]
    dst = edge_index[1]
    nid = jnp.arange(n, dtype=jnp.int32)
    keys_all = jnp.concatenate([rel_types * n + src, nr * n + nid])
    dst_all = jnp.concatenate([dst, nid])
    e_tot = keys_all.shape[0]

    nch1 = _round_up(-(-e_tot // (_NS * _K)), _NBUF)
    k2 = keys_all * 2
    keys1 = _pad_reshape(jnp.stack([k2, k2 + 1]), e_tot, nch1, 0)
    dst1 = _pad_reshape(jnp.stack([dst_all, dst_all]), e_tot, nch1, _PAD_DST)

    eh = -(-e_tot // 2)
    nch2 = _round_up(-(-eh // (_NS * _K)), _NBUF)
    pad_tail = 2 * eh - e_tot
    keys_h = jnp.pad(keys_all, (0, pad_tail)).reshape(2, eh)
    dst_h = jnp.pad(dst_all, (0, pad_tail), constant_values=_PAD_DST).reshape(2, eh)
    keys2 = _pad_reshape(keys_h, eh, nch2, 0)
    dst2 = _pad_reshape(dst_h, eh, nch2, _PAD_DST)

    zeros = jnp.zeros((_NROWS, 128), jnp.float32)

    proj1 = _stacked_mm(h, w1_all)
    table1 = proj1.reshape((nr + 1) * n * 2, 128)
    agg1 = _make_sc_accum(nch1, table1.shape[0])(table1, keys1, dst1, zeros)
    h1 = _combine1(agg1, b1.reshape(1, 256))

    proj2 = _stacked_mm(h1, w2_all)
    table2 = proj2.reshape((nr + 1) * n, 128)
    agg2 = _make_sc_accum(nch2, table2.shape[0])(table2, keys2, dst2, zeros)

    return _final(agg2, b2.reshape(1, 128), Wc, bc.reshape(1, Wc.shape[1]))


# speedup vs baseline: 9.3413x; 1.0906x over previous
import functools

import jax
import jax.numpy as jnp
from jax import lax
from jax.experimental import pallas as pl
from jax.experimental.pallas import tpu as pltpu
from jax.experimental.pallas import tpu_sc as plsc

_NC = 2
_NS = 16
_K = 128

_N = 10000
_NROWS = 10112
_ZR = _NROWS // _NS
_PAD_DST = _N
_OR_HI = 632
_OR_LO = _N - 15 * _OR_HI


_NBUF = 3


@functools.lru_cache(maxsize=None)
def _make_sc_accum(n_chunks, table_rows):
    assert n_chunks % _NBUF == 0
    mesh = plsc.VectorSubcoreMesh(core_axis_name="c", subcore_axis_name="s")

    scratch = []
    for _ in range(_NBUF):
        scratch += [
            pltpu.VMEM((_K,), jnp.int32),
            pltpu.VMEM((_K,), jnp.int32),
            pltpu.VMEM((_K, 128), jnp.float32),
            pltpu.SemaphoreType.DMA,
            pltpu.SemaphoreType.DMA,
        ]
    scratch.append(pltpu.VMEM_SHARED((_NROWS, 128), jnp.float32))

    @functools.partial(
        pl.kernel,
        out_type=jax.ShapeDtypeStruct((_NC, _N, 128), jnp.float32),
        mesh=mesh,
        scratch_types=scratch,
    )
    def sc_accum(table, keys, dsts, zeros, out, *refs):
        acc = refs[-1]
        kv = [refs[5 * b + 0] for b in range(_NBUF)]
        dv = [refs[5 * b + 1] for b in range(_NBUF)]
        rv = [refs[5 * b + 2] for b in range(_NBUF)]
        gsem = [refs[5 * b + 3] for b in range(_NBUF)]
        ssem = [refs[5 * b + 4] for b in range(_NBUF)]
        c = lax.axis_index("c")
        s = lax.axis_index("s")
        pltpu.sync_copy(zeros.at[pl.ds(s * _ZR, _ZR)], acc.at[pl.ds(s * _ZR, _ZR)])
        plsc.subcore_barrier()

        tile_base = (c * _NS + s) * n_chunks * _K

        for b in range(2):
            off = tile_base + b * _K
            pltpu.sync_copy(keys.at[pl.ds(off, _K)], kv[b])
            pltpu.sync_copy(dsts.at[pl.ds(off, _K)], dv[b])
            pltpu.async_copy(table.at[kv[b]], rv[b], gsem[b])

        @pl.loop(0, n_chunks, step=_NBUF)
        def _chunk(j):
            for b in range(_NBUF):
                cur = j + b
                pltpu.make_async_copy(table.at[kv[b]], rv[b], gsem[b]).wait()
                pltpu.async_copy(rv[b], acc.at[dv[b]], ssem[b], add=True)
                bb = (b + 2) % _NBUF
                nxt = cur + 2

                @pl.when(nxt < n_chunks)
                def _prefetch():
                    @pl.when(nxt >= _NBUF)
                    def _free():
                        pltpu.make_async_copy(rv[bb], acc.at[dv[bb]],
                                              ssem[bb]).wait()

                    off = tile_base + nxt * _K
                    pltpu.sync_copy(keys.at[pl.ds(off, _K)], kv[bb])
                    pltpu.sync_copy(dsts.at[pl.ds(off, _K)], dv[bb])
                    pltpu.async_copy(table.at[kv[bb]], rv[bb], gsem[bb])

        for b in range(_NBUF):
            pltpu.make_async_copy(rv[b], acc.at[dv[b]], ssem[b]).wait()

        plsc.subcore_barrier()

        @pl.when(s < _NS - 1)
        def _copy_hi():
            pltpu.sync_copy(acc.at[pl.ds(s * _OR_HI, _OR_HI)],
                            out.at[c, pl.ds(s * _OR_HI, _OR_HI)])

        @pl.when(s == _NS - 1)
        def _copy_lo():
            pltpu.sync_copy(acc.at[pl.ds(15 * _OR_HI, _OR_LO)],
                            out.at[c, pl.ds(15 * _OR_HI, _OR_LO)])

    return sc_accum


def _mm_body(x_ref, w_ref, o_ref):
    o_ref[0] = jnp.dot(x_ref[...], w_ref[0], preferred_element_type=jnp.float32)


def _stacked_mm(x, w):
    g, din, dout = w.shape
    n = x.shape[0]
    bm = 1000
    return pl.pallas_call(
        _mm_body,
        grid=(g, n // bm),
        in_specs=[
            pl.BlockSpec((bm, din), lambda gi, m: (m, 0)),
            pl.BlockSpec((1, din, dout), lambda gi, m: (gi, 0, 0)),
        ],
        out_specs=pl.BlockSpec((1, bm, dout), lambda gi, m: (gi, m, 0)),
        out_shape=jax.ShapeDtypeStruct((g, n, dout), jnp.float32),
    )(x, w)


def _combine1_body(a_ref, b_ref, o_ref):
    bv = b_ref[...]
    o_ref[:, 0:128] = jnp.maximum(a_ref[0] + bv[:, 0:128], 0.0)
    o_ref[:, 128:256] = jnp.maximum(a_ref[1] + bv[:, 128:256], 0.0)


def _combine1(agg, b):
    bm = 1000
    return pl.pallas_call(
        _combine1_body,
        grid=(_N // bm,),
        in_specs=[
            pl.BlockSpec((2, bm, 128), lambda m: (0, m, 0)),
            pl.BlockSpec((1, 256), lambda m: (0, 0)),
        ],
        out_specs=pl.BlockSpec((bm, 256), lambda m: (m, 0)),
        out_shape=jax.ShapeDtypeStruct((_N, 256), jnp.float32),
    )(agg, b)


def _final_body(a_ref, b_ref, wc_ref, bc_ref, o_ref):
    h2 = jnp.maximum(a_ref[0] + a_ref[1] + b_ref[...], 0.0)
    m = jnp.sum(h2, axis=0, keepdims=True) * (1.0 / _N)
    logits = jnp.dot(m, wc_ref[...], preferred_element_type=jnp.float32) + bc_ref[...]
    z = logits - jnp.max(logits, axis=1, keepdims=True)
    e = jnp.exp(z)
    o_ref[...] = e / jnp.sum(e, axis=1, keepdims=True)


def _final(agg, b2, wc, bc):
    c = wc.shape[1]
    return pl.pallas_call(
        _final_body,
        out_shape=jax.ShapeDtypeStruct((1, c), jnp.float32),
    )(agg, b2, wc, bc)


def _round_up(x, m):
    return -(-x // m) * m


def _pad_reshape(arr, per_core, n_chunks, pad_value):
    target = _NS * n_chunks * _K
    arr = jnp.pad(arr, ((0, 0), (0, target - per_core)), constant_values=pad_value)
    return arr.reshape(-1)


def kernel(h, edge_index, rel_types, W1, W1_self, b1, W2, W2_self, b2, Wc, bc):
    h = h.astype(jnp.float32)
    n = h.shape[0]
    nr = W1.shape[0]

    w1_all = jnp.concatenate([W1, W1_self[None]], axis=0)
    w2_all = jnp.concatenate([W2, W2_self[None]], axis=0)

    src = edge_index[0]
    dst = edge_index[1]
    nid = jnp.arange(n, dtype=jnp.int32)
    keys_all = jnp.concatenate([rel_types * n + src, nr * n + nid])
    dst_all = jnp.concatenate([dst, nid])
    e_tot = keys_all.shape[0]

    nch1 = _round_up(-(-e_tot // (_NS * _K)), _NBUF)
    k2 = keys_all * 2
    keys1 = _pad_reshape(jnp.stack([k2, k2 + 1]), e_tot, nch1, 0)
    dst1 = _pad_reshape(jnp.stack([dst_all, dst_all]), e_tot, nch1, _PAD_DST)

    eh = -(-e_tot // 2)
    nch2 = _round_up(-(-eh // (_NS * _K)), _NBUF)
    pad_tail = 2 * eh - e_tot
    keys_h = jnp.pad(keys_all, (0, pad_tail)).reshape(2, eh)
    dst_h = jnp.pad(dst_all, (0, pad_tail), constant_values=_PAD_DST).reshape(2, eh)
    keys2 = _pad_reshape(keys_h, eh, nch2, 0)
    dst2 = _pad_reshape(dst_h, eh, nch2, _PAD_DST)

    zeros = jnp.zeros((_NROWS, 128), jnp.float32)

    proj1 = _stacked_mm(h, w1_all)
    table1 = proj1.reshape((nr + 1) * n * 2, 128)
    agg1 = _make_sc_accum(nch1, table1.shape[0])(table1, keys1, dst1, zeros)
    h1 = _combine1(agg1, b1.reshape(1, 256))

    proj2 = _stacked_mm(h1, w2_all)
    table2 = proj2.reshape((nr + 1) * n, 128)
    agg2 = _make_sc_accum(nch2, table2.shape[0])(table2, keys2, dst2, zeros)

    return _final(agg2, b2.reshape(1, 128), Wc, bc.reshape(1, Wc.shape[1]))


# speedup vs baseline: 9.5776x; 1.0253x over previous
import functools

import jax
import jax.numpy as jnp
from jax import lax
from jax.experimental import pallas as pl
from jax.experimental.pallas import tpu as pltpu
from jax.experimental.pallas import tpu_sc as plsc

_NC = 2
_NS = 16
_K = 128

_N = 10000
_NROWS = 10112
_ZR = _NROWS // _NS
_PAD_DST = _N
_OR_HI = 632
_OR_LO = _N - 15 * _OR_HI


_NBUF = 3


@functools.lru_cache(maxsize=None)
def _make_sc_accum(n_chunks, table_rows):
    assert n_chunks % _NBUF == 0
    mesh = plsc.VectorSubcoreMesh(core_axis_name="c", subcore_axis_name="s")

    scratch = []
    for _ in range(_NBUF):
        scratch += [
            pltpu.VMEM((_K,), jnp.int32),
            pltpu.VMEM((_K,), jnp.int32),
            pltpu.VMEM((_K, 128), jnp.float32),
            pltpu.SemaphoreType.DMA,
            pltpu.SemaphoreType.DMA,
        ]
    scratch.append(pltpu.VMEM_SHARED((_NROWS, 128), jnp.float32))

    @functools.partial(
        pl.kernel,
        out_type=jax.ShapeDtypeStruct((_NC, _N, 128), jnp.float32),
        mesh=mesh,
        scratch_types=scratch,
    )
    def sc_accum(table, keys, dsts, zeros, out, *refs):
        acc = refs[-1]
        kv = [refs[5 * b + 0] for b in range(_NBUF)]
        dv = [refs[5 * b + 1] for b in range(_NBUF)]
        rv = [refs[5 * b + 2] for b in range(_NBUF)]
        gsem = [refs[5 * b + 3] for b in range(_NBUF)]
        ssem = [refs[5 * b + 4] for b in range(_NBUF)]
        c = lax.axis_index("c")
        s = lax.axis_index("s")
        pltpu.sync_copy(zeros.at[pl.ds(s * _ZR, _ZR)], acc.at[pl.ds(s * _ZR, _ZR)])
        plsc.subcore_barrier()

        tile_base = (c * _NS + s) * n_chunks * _K

        for b in range(2):
            off = tile_base + b * _K
            pltpu.sync_copy(keys.at[pl.ds(off, _K)], kv[b])
            pltpu.sync_copy(dsts.at[pl.ds(off, _K)], dv[b])
            pltpu.async_copy(table.at[kv[b]], rv[b], gsem[b])

        @pl.loop(0, n_chunks, step=_NBUF)
        def _chunk(j):
            for b in range(_NBUF):
                cur = j + b
                pltpu.make_async_copy(table.at[kv[b]], rv[b], gsem[b]).wait()
                pltpu.async_copy(rv[b], acc.at[dv[b]], ssem[b], add=True)
                bb = (b + 2) % _NBUF
                nxt = cur + 2

                @pl.when(nxt < n_chunks)
                def _prefetch():
                    @pl.when(nxt >= _NBUF)
                    def _free():
                        pltpu.make_async_copy(rv[bb], acc.at[dv[bb]],
                                              ssem[bb]).wait()

                    off = tile_base + nxt * _K
                    pltpu.sync_copy(keys.at[pl.ds(off, _K)], kv[bb])
                    pltpu.sync_copy(dsts.at[pl.ds(off, _K)], dv[bb])
                    pltpu.async_copy(table.at[kv[bb]], rv[bb], gsem[bb])

        for b in range(_NBUF):
            pltpu.make_async_copy(rv[b], acc.at[dv[b]], ssem[b]).wait()

        plsc.subcore_barrier()

        @pl.when(s < _NS - 1)
        def _copy_hi():
            pltpu.sync_copy(acc.at[pl.ds(s * _OR_HI, _OR_HI)],
                            out.at[c, pl.ds(s * _OR_HI, _OR_HI)])

        @pl.when(s == _NS - 1)
        def _copy_lo():
            pltpu.sync_copy(acc.at[pl.ds(15 * _OR_HI, _OR_LO)],
                            out.at[c, pl.ds(15 * _OR_HI, _OR_LO)])

    return sc_accum


def _mm_body(x_ref, w_ref, o_ref):
    o_ref[0] = jnp.dot(x_ref[...], w_ref[0], preferred_element_type=jnp.float32)


def _stacked_mm(x, w):
    g, din, dout = w.shape
    n = x.shape[0]
    bm = 1000
    return pl.pallas_call(
        _mm_body,
        grid=(g, n // bm),
        in_specs=[
            pl.BlockSpec((bm, din), lambda gi, m: (m, 0)),
            pl.BlockSpec((1, din, dout), lambda gi, m: (gi, 0, 0)),
        ],
        out_specs=pl.BlockSpec((1, bm, dout), lambda gi, m: (gi, m, 0)),
        out_shape=jax.ShapeDtypeStruct((g, n, dout), jnp.float32),
    )(x, w)


def _relu_mm_body(a_ref, b_ref, w_ref, o_ref, h1_ref):
    @pl.when(pl.program_id(1) == 0)
    def _build():
        bv = b_ref[...]
        h1_ref[:, 0:128] = jnp.maximum(a_ref[0] + bv[:, 0:128], 0.0)
        h1_ref[:, 128:256] = jnp.maximum(a_ref[1] + bv[:, 128:256], 0.0)

    o_ref[0] = jnp.dot(h1_ref[...], w_ref[0], preferred_element_type=jnp.float32)


def _relu_stacked_mm(agg, b, w):
    g, din, dout = w.shape
    bm = 1000
    return pl.pallas_call(
        _relu_mm_body,
        grid=(_N // bm, g),
        in_specs=[
            pl.BlockSpec((2, bm, 128), lambda m, gi: (0, m, 0)),
            pl.BlockSpec((1, 256), lambda m, gi: (0, 0)),
            pl.BlockSpec((1, din, dout), lambda m, gi: (gi, 0, 0)),
        ],
        out_specs=pl.BlockSpec((1, bm, dout), lambda m, gi: (gi, m, 0)),
        out_shape=jax.ShapeDtypeStruct((g, _N, dout), jnp.float32),
        scratch_shapes=[pltpu.VMEM((bm, 256), jnp.float32)],
    )(agg, b, w)


def _final_body(a_ref, b_ref, wc_ref, bc_ref, o_ref):
    h2 = jnp.maximum(a_ref[0] + a_ref[1] + b_ref[...], 0.0)
    m = jnp.sum(h2, axis=0, keepdims=True) * (1.0 / _N)
    logits = jnp.dot(m, wc_ref[...], preferred_element_type=jnp.float32) + bc_ref[...]
    z = logits - jnp.max(logits, axis=1, keepdims=True)
    e = jnp.exp(z)
    o_ref[...] = e / jnp.sum(e, axis=1, keepdims=True)


def _final(agg, b2, wc, bc):
    c = wc.shape[1]
    return pl.pallas_call(
        _final_body,
        out_shape=jax.ShapeDtypeStruct((1, c), jnp.float32),
    )(agg, b2, wc, bc)


def _round_up(x, m):
    return -(-x // m) * m


def _pad_reshape(arr, per_core, n_chunks, pad_value):
    target = _NS * n_chunks * _K
    arr = jnp.pad(arr, ((0, 0), (0, target - per_core)), constant_values=pad_value)
    return arr.reshape(-1)


def kernel(h, edge_index, rel_types, W1, W1_self, b1, W2, W2_self, b2, Wc, bc):
    h = h.astype(jnp.float32)
    n = h.shape[0]
    nr = W1.shape[0]

    w1_all = jnp.concatenate([W1, W1_self[None]], axis=0)
    w2_all = jnp.concatenate([W2, W2_self[None]], axis=0)

    src = edge_index[0]
    dst = edge_index[1]
    nid = jnp.arange(n, dtype=jnp.int32)
    keys_all = jnp.concatenate([rel_types * n + src, nr * n + nid])
    dst_all = jnp.concatenate([dst, nid])
    e_tot = keys_all.shape[0]

    nch1 = _round_up(-(-e_tot // (_NS * _K)), _NBUF)
    k2 = keys_all * 2
    keys1 = _pad_reshape(jnp.stack([k2, k2 + 1]), e_tot, nch1, 0)
    dst1 = _pad_reshape(jnp.stack([dst_all, dst_all]), e_tot, nch1, _PAD_DST)

    eh = -(-e_tot // 2)
    nch2 = _round_up(-(-eh // (_NS * _K)), _NBUF)
    pad_tail = 2 * eh - e_tot
    keys_h = jnp.pad(keys_all, (0, pad_tail)).reshape(2, eh)
    dst_h = jnp.pad(dst_all, (0, pad_tail), constant_values=_PAD_DST).reshape(2, eh)
    keys2 = _pad_reshape(keys_h, eh, nch2, 0)
    dst2 = _pad_reshape(dst_h, eh, nch2, _PAD_DST)

    zeros = jnp.zeros((_NROWS, 128), jnp.float32)

    proj1 = _stacked_mm(h, w1_all)
    table1 = proj1.reshape((nr + 1) * n * 2, 128)
    agg1 = _make_sc_accum(nch1, table1.shape[0])(table1, keys1, dst1, zeros)

    proj2 = _relu_stacked_mm(agg1, b1.reshape(1, 256), w2_all)
    table2 = proj2.reshape((nr + 1) * n, 128)
    agg2 = _make_sc_accum(nch2, table2.shape[0])(table2, keys2, dst2, zeros)

    return _final(agg2, b2.reshape(1, 128), Wc, bc.reshape(1, Wc.shape[1]))
